# Initial kernel scaffold; baseline (speedup 1.0000x reference)
import jax
import jax.numpy as jnp
from jax.experimental import pallas as pl


def kernel(x, edge_index, W1, b1, g1, beta1, W2, b2, g2, beta2):
    raise NotImplementedError("write your pallas kernel here")


# speedup vs baseline: 27.2229x; 27.2229x over previous
import functools

import jax
import jax.numpy as jnp
from jax import lax
from jax.experimental import pallas as pl
from jax.experimental.pallas import tpu as pltpu
from jax.experimental.pallas import tpu_sc as plsc

_NC = 2
_NS = 16
_NW = _NC * _NS
_LN = 16
_CH = 128
_BLK = 8
_HW = 16
_BR = 1000


def _sc_mesh():
    return plsc.VectorSubcoreMesh(
        core_axis_name="c", subcore_axis_name="s",
        num_cores=_NC, num_subcores=_NS)


def _pad_rows(n):
    q = _NS * 8
    return (n + q - 1) // q * q


def _nch(e):
    ew = e // _NW
    return (ew + _CH - 1) // _CH


@functools.lru_cache(maxsize=None)
def _deg_kernel(n, e):
    nch = _nch(e)
    npad = _pad_rows(n)
    rpt = npad // _NS
    nzc = rpt // _CH

    def body(cols_hbm, out_hbm, cols_v, ones_v, zero_v, hist):
        c = lax.axis_index("c")
        s = lax.axis_index("s")
        wid = c * _NS + s
        pltpu.sync_copy(cols_hbm.at[wid], cols_v)
        one = jnp.full((_LN,), 1.0, jnp.float32)
        zero = jnp.zeros((_LN,), jnp.float32)

        def fill(r, _):
            ones_v[r, :] = one
            zero_v[r, :] = zero
            return 0
        lax.fori_loop(0, _CH, fill, 0)
        for b in range(nzc):
            pltpu.sync_copy(zero_v, hist.at[pl.ds(s * rpt + b * _CH, _CH)])
        plsc.subcore_barrier()

        def chunk(j, _):
            pltpu.sync_copy(ones_v, hist.at[cols_v.at[j]], add=True)
            return 0
        lax.fori_loop(0, nch, chunk, 0)
        plsc.subcore_barrier()
        pltpu.sync_copy(hist.at[pl.ds(s * rpt, rpt)],
                        out_hbm.at[c, pl.ds(s * rpt, rpt)])

    return pl.kernel(
        body,
        out_type=jax.ShapeDtypeStruct((_NC, npad, _HW), jnp.float32),
        mesh=_sc_mesh(),
        scratch_types=[
            pltpu.VMEM((nch, _CH), jnp.int32),
            pltpu.VMEM((_CH, _HW), jnp.float32),
            pltpu.VMEM((_CH, _HW), jnp.float32),
            pltpu.VMEM_SHARED((npad, _HW), jnp.float32),
        ],
    )


@functools.lru_cache(maxsize=None)
def _scatter_kernel(n, d, e):
    nch = _nch(e)
    nblk = nch // _BLK
    npad = _pad_rows(n)
    rpt = npad // _NS
    nzc = rpt // _CH

    def body(g_hbm, rows_hbm, cols_hbm, out_hbm,
             rows_v, cols_v, bufa, bufb, acc, sema, semb):
        c = lax.axis_index("c")
        s = lax.axis_index("s")
        wid = c * _NS + s
        zero = jnp.zeros((_LN,), jnp.float32)

        def fill(r, _):
            for g in range(d // _LN):
                bufa[r, pl.ds(g * _LN, _LN)] = zero
            return 0
        lax.fori_loop(0, _CH, fill, 0)
        for b in range(nzc):
            pltpu.sync_copy(bufa, acc.at[pl.ds(s * rpt + b * _CH, _CH)])
        plsc.subcore_barrier()

        def block(bb, _):
            pltpu.sync_copy(rows_hbm.at[wid, pl.ds(bb * _BLK, _BLK)], rows_v)
            pltpu.sync_copy(cols_hbm.at[wid, pl.ds(bb * _BLK, _BLK)], cols_v)
            cps = [None, None]
            cps[0] = pltpu.async_copy(g_hbm.at[rows_v.at[0]], bufa, sema)
            for k in range(_BLK):
                cur, sem = (bufa, sema) if k % 2 == 0 else (bufb, semb)
                if k + 1 < _BLK:
                    nxt, nsem = (bufb, semb) if k % 2 == 0 else (bufa, sema)
                    cps[1] = pltpu.async_copy(
                        g_hbm.at[rows_v.at[k + 1]], nxt, nsem)
                cps[0].wait()
                cps[0] = cps[1]
                pltpu.sync_copy(cur, acc.at[cols_v.at[k]], add=True)
            return 0
        lax.fori_loop(0, nblk, block, 0)
        plsc.subcore_barrier()
        pltpu.sync_copy(acc.at[pl.ds(s * rpt, rpt)],
                        out_hbm.at[c, pl.ds(s * rpt, rpt)])

    return pl.kernel(
        body,
        out_type=jax.ShapeDtypeStruct((_NC, npad, d), jnp.float32),
        mesh=_sc_mesh(),
        scratch_types=[
            pltpu.VMEM((_BLK, _CH), jnp.int32),
            pltpu.VMEM((_BLK, _CH), jnp.int32),
            pltpu.VMEM((_CH, d), jnp.float32),
            pltpu.VMEM((_CH, d), jnp.float32),
            pltpu.VMEM_SHARED((npad, d), jnp.float32),
            pltpu.SemaphoreType.DMA,
            pltpu.SemaphoreType.DMA,
        ],
    )


def _dvec(hist_ref):
    deg = hist_ref[0, :, 0:1] + hist_ref[1, :, 0:1] + 1.0
    return lax.rsqrt(deg)


def _matmul_t(a, w):
    return lax.dot_general(a, w, (((1,), (1,)), ((), ())),
                           preferred_element_type=jnp.float32,
                           precision=lax.Precision.HIGHEST)


def _ln_gelu(z, gam, bet):
    mu = jnp.mean(z, axis=-1, keepdims=True)
    zc = z - mu
    var = jnp.mean(zc * zc, axis=-1, keepdims=True)
    zn = zc * lax.rsqrt(var + 1e-5) * gam + bet
    return 0.5 * zn * (1.0 + lax.erf(zn * 0.7071067811865476))


def _tc1_body(x_ref, w1_ref, hist_ref, o_ref):
    dv = _dvec(hist_ref)
    o_ref[...] = dv * _matmul_t(x_ref[...], w1_ref[...])


def _tc2_body(acc_ref, g_ref, hist_ref, b_ref, gam_ref, bet_ref, w2_ref, o_ref):
    dv = _dvec(hist_ref)
    z = dv * (acc_ref[0] + acc_ref[1] + g_ref[...]) + b_ref[...]
    a = _ln_gelu(z, gam_ref[...], bet_ref[...])
    o_ref[...] = dv * _matmul_t(a, w2_ref[...])


def _tc3_body(acc_ref, g_ref, hist_ref, b_ref, gam_ref, bet_ref, x_ref, o_ref):
    dv = _dvec(hist_ref)
    z = dv * (acc_ref[0] + acc_ref[1] + g_ref[...]) + b_ref[...]
    o_ref[...] = _ln_gelu(z, gam_ref[...], bet_ref[...]) + x_ref[...]


def _row_spec(d):
    return pl.BlockSpec((_BR, d), lambda i: (i, 0))


def _full_spec(shape):
    nd = len(shape)
    return pl.BlockSpec(shape, lambda i, _n=nd: (0,) * _n)


def _part_spec(d):
    return pl.BlockSpec((_NC, _BR, d), lambda i: (0, i, 0))


@functools.lru_cache(maxsize=None)
def _tc1_call(n, d):
    return pl.pallas_call(
        _tc1_body,
        grid=(n // _BR,),
        in_specs=[_row_spec(d), _full_spec((d, d)), _part_spec(_HW)],
        out_specs=_row_spec(d),
        out_shape=jax.ShapeDtypeStruct((n, d), jnp.float32),
    )


@functools.lru_cache(maxsize=None)
def _tc2_call(n, d):
    return pl.pallas_call(
        _tc2_body,
        grid=(n // _BR,),
        in_specs=[_part_spec(d), _row_spec(d), _part_spec(_HW),
                  _full_spec((1, d)), _full_spec((1, d)), _full_spec((1, d)),
                  _full_spec((d, d))],
        out_specs=_row_spec(d),
        out_shape=jax.ShapeDtypeStruct((n, d), jnp.float32),
    )


@functools.lru_cache(maxsize=None)
def _tc3_call(n, d):
    return pl.pallas_call(
        _tc3_body,
        grid=(n // _BR,),
        in_specs=[_part_spec(d), _row_spec(d), _part_spec(_HW),
                  _full_spec((1, d)), _full_spec((1, d)), _full_spec((1, d)),
                  _row_spec(d)],
        out_specs=_row_spec(d),
        out_shape=jax.ShapeDtypeStruct((n, d), jnp.float32),
    )


def _pad_edges(edge_index, n, e):
    nch = _nch(e)
    npad = _pad_rows(n)
    ew = e // _NW
    padcnt = nch * _CH - ew
    rows2 = edge_index[0].reshape(_NW, ew)
    cols2 = edge_index[1].reshape(_NW, ew)
    if padcnt:
        prow = jnp.zeros((_NW, padcnt), jnp.int32)
        pcol = jnp.broadcast_to(
            n + (jnp.arange(padcnt, dtype=jnp.int32) % (npad - n)),
            (_NW, padcnt))
        rows2 = jnp.concatenate([rows2, prow], axis=1)
        cols2 = jnp.concatenate([cols2, pcol], axis=1)
    return (rows2.reshape(_NW, nch, _CH), cols2.reshape(_NW, nch, _CH))


def kernel(x, edge_index, W1, b1, g1, beta1, W2, b2, g2, beta2):
    n, d = x.shape
    e = edge_index.shape[1]
    rows3, cols3 = _pad_edges(edge_index, n, e)

    hist = _deg_kernel(n, e)(cols3)

    b1r = b1.reshape(1, d)
    gam1 = g1.reshape(1, d)
    bet1 = beta1.reshape(1, d)
    b2r = b2.reshape(1, d)
    gam2 = g2.reshape(1, d)
    bet2 = beta2.reshape(1, d)

    scat = _scatter_kernel(n, d, e)
    g1m = _tc1_call(n, d)(x, W1, hist)
    acc1 = scat(g1m, rows3, cols3)
    g2m = _tc2_call(n, d)(acc1, g1m, hist, b1r, gam1, bet1, W2)
    acc2 = scat(g2m, rows3, cols3)
    out = _tc3_call(n, d)(acc2, g2m, hist, b2r, gam2, bet2, x)
    return out
